# Initial kernel scaffold; baseline (speedup 1.0000x reference)
import jax
import jax.numpy as jnp
from jax.experimental import pallas as pl


def kernel(input, state, weights, biases, src, dst, act_ids, output_ids):
    raise NotImplementedError("write your pallas kernel here")


# speedup vs baseline: 58.5592x; 58.5592x over previous
import functools

import jax
import jax.numpy as jnp
from jax import lax
from jax.experimental import pallas as pl
from jax.experimental.pallas import tpu as pltpu
from jax.experimental.pallas import tpu_sc as plsc

_N = 10000
_E = 320000
_REFRACTORY = 0.33
_RELU_CLIP = 1.0
_NW = 32
_EPW = _E // _NW
_L = 16


def _sc_edge_kernel(state_hbm, w_hbm, src_hbm, dst_hbm, oid_hbm, bias_hbm,
                    act_hbm, part_out, base_out, actv_out,
                    state_v, src_v, dst_v, w_v, row_v, oid_v, bias_v, act_v,
                    base_v, acti_v):
    wid = lax.axis_index("s") * 2 + lax.axis_index("c")
    pltpu.sync_copy(state_hbm, state_v)
    pltpu.sync_copy(oid_hbm, oid_v)
    pltpu.sync_copy(src_hbm.at[pl.ds(wid * _EPW, _EPW)], src_v)
    pltpu.sync_copy(dst_hbm.at[pl.ds(wid * _EPW, _EPW)], dst_v)
    pltpu.sync_copy(w_hbm.at[pl.ds(wid * _EPW, _EPW)], w_v)

    lane = lax.iota(jnp.int32, _L)
    o0 = plsc.load_gather(oid_v, [jnp.full((_L,), 0, jnp.int32)])
    o1 = plsc.load_gather(oid_v, [jnp.full((_L,), 1, jnp.int32)])
    o2 = plsc.load_gather(oid_v, [jnp.full((_L,), 2, jnp.int32)])
    o3 = plsc.load_gather(oid_v, [jnp.full((_L,), 3, jnp.int32)])
    zero = jnp.zeros((_L,), jnp.float32)

    def body(i, carry):
        a0, a1, a2, a3 = carry
        sl = pl.ds(i * _L, _L)
        d = dst_v[sl]
        s = src_v[sl]
        w = w_v[sl]
        m = w * plsc.load_gather(state_v, [s])
        a0 = a0 + jnp.where(d == o0, m, zero)
        a1 = a1 + jnp.where(d == o1, m, zero)
        a2 = a2 + jnp.where(d == o2, m, zero)
        a3 = a3 + jnp.where(d == o3, m, zero)
        return a0, a1, a2, a3

    a0, a1, a2, a3 = lax.fori_loop(0, _EPW // _L, body,
                                   (zero, zero, zero, zero))
    t0, t1, t2, t3 = jnp.sum(a0), jnp.sum(a1), jnp.sum(a2), jnp.sum(a3)
    row = jnp.where(lane == 0, t0,
          jnp.where(lane == 1, t1,
          jnp.where(lane == 2, t2,
          jnp.where(lane == 3, t3, 0.0)))) * _REFRACTORY
    row_v[...] = row
    pltpu.sync_copy(row_v, part_out.at[wid])

    @pl.when(wid == 0)
    def _():
        pltpu.sync_copy(bias_hbm, bias_v)
        pltpu.sync_copy(act_hbm, act_v)
        oid_vec = plsc.load_gather(oid_v, [jnp.minimum(lane, 3)])
        pv = plsc.load_gather(state_v, [oid_vec]) * _REFRACTORY
        bv = plsc.load_gather(bias_v, [oid_vec])
        base_v[...] = pv + bv
        acti_v[...] = plsc.load_gather(act_v, [oid_vec])
        pltpu.sync_copy(base_v, base_out.at[0])
        pltpu.sync_copy(acti_v, actv_out.at[0])


_sc_edge_call = functools.partial(
    pl.kernel,
    mesh=plsc.VectorSubcoreMesh(core_axis_name="c", subcore_axis_name="s"),
    compiler_params=pltpu.CompilerParams(needs_layout_passes=False),
    out_type=[
        jax.ShapeDtypeStruct((_NW, _L), jnp.float32),
        jax.ShapeDtypeStruct((1, _L), jnp.float32),
        jax.ShapeDtypeStruct((1, _L), jnp.int32),
    ],
    scratch_types=[
        pltpu.VMEM((_N,), jnp.float32),
        pltpu.VMEM((_EPW,), jnp.int32),
        pltpu.VMEM((_EPW,), jnp.int32),
        pltpu.VMEM((_EPW,), jnp.float32),
        pltpu.VMEM((_L,), jnp.float32),
        pltpu.VMEM((_L,), jnp.int32),
        pltpu.VMEM((_N,), jnp.float32),
        pltpu.VMEM((_N,), jnp.int32),
        pltpu.VMEM((_L,), jnp.float32),
        pltpu.VMEM((_L,), jnp.int32),
    ],
)(_sc_edge_kernel)


def _tc_finish_kernel(part_ref, base_ref, act_ref, out_ref):
    x = jnp.sum(part_ref[...], axis=0, keepdims=True) + base_ref[...]
    a = act_ref[...]
    r = x
    r = jnp.where(a == 1, jnp.maximum(x, 0.0), r)
    r = jnp.where(a == 2, jnp.where(x >= 0, x, 0.01 * x), r)
    r = jnp.where(a == 3, jnp.clip(x, 0.0, _RELU_CLIP), r)
    r = jnp.where(a == 4, jnp.tanh(x), r)
    r = jnp.where(a == 5, jax.nn.sigmoid(x), r)
    r = jnp.where(a == 6, jnp.maximum(x, 0.0) + jnp.log1p(jnp.exp(-jnp.abs(x))), r)
    r = jnp.where(a == 7, jnp.abs(x), r)
    out_ref[...] = r


def kernel(input, state, weights, biases, src, dst, act_ids, output_ids):
    del input
    src = src.astype(jnp.int32)
    dst = dst.astype(jnp.int32)
    oid16 = jnp.concatenate(
        [output_ids.astype(jnp.int32),
         jnp.zeros((_L - output_ids.shape[0],), jnp.int32)])
    part, base, actv = _sc_edge_call(
        state, weights, src, dst, oid16, biases, act_ids)
    res = pl.pallas_call(
        _tc_finish_kernel,
        out_shape=jax.ShapeDtypeStruct((1, _L), jnp.float32),
    )(part, base, actv)
    return res[0, :4]
